# Initial kernel scaffold; baseline (speedup 1.0000x reference)
import jax
import jax.numpy as jnp
from jax.experimental import pallas as pl


def kernel(node_types, trust_levels, providers, node_attrs, type_table, trust_table, provider_table, attr_W, attr_b):
    raise NotImplementedError("write your pallas kernel here")


# speedup vs baseline: 1.1947x; 1.1947x over previous
import functools

import jax
import jax.numpy as jnp
from jax import lax
from jax.experimental import pallas as pl
from jax.experimental.pallas import tpu as pltpu
from jax.experimental.pallas import tpu_sc as plsc

DIM = 128
PROVIDER_DIM = 64
B, N = 4096, 50
R = B * N
NC, NS = 2, 16
NW = NC * NS
RPW = R // NW
GC = 128
NCH = RPW // GC

_mesh = plsc.VectorSubcoreMesh(
    core_axis_name="c", subcore_axis_name="s", num_cores=NC, num_subcores=NS)


@functools.partial(
    pl.kernel,
    out_type=jax.ShapeDtypeStruct((R, 3, DIM), jnp.float32),
    mesh=_mesh,
    scratch_types=[
        pltpu.VMEM((RPW,), jnp.int32),
        pltpu.VMEM((RPW,), jnp.int32),
        pltpu.VMEM((RPW,), jnp.int32),
        pltpu.VMEM((GC, 1, DIM), jnp.float32),
        pltpu.SemaphoreType.DMA,
    ],
)
def _encode(types_hbm, trust_hbm, prov_hbm, ttab_hbm, rtab_hbm, ptab_hbm,
            out_hbm, t_v, r_v, p_v, rows_v, sem):
    wid = lax.axis_index("s") * NC + lax.axis_index("c")
    base = wid * RPW
    pltpu.sync_copy(types_hbm.at[pl.ds(base, RPW)], t_v)
    pltpu.sync_copy(trust_hbm.at[pl.ds(base, RPW)], r_v)
    pltpu.sync_copy(prov_hbm.at[pl.ds(base, RPW)], p_v)

    def chunk(i, carry):
        rows = pl.ds(base + i * GC, GC)
        sl = pl.ds(i * GC, GC)
        pltpu.async_copy(ttab_hbm.at[t_v.at[sl]], rows_v, sem).wait()
        pltpu.sync_copy(rows_v, out_hbm.at[rows, pl.ds(0, 1)])
        pltpu.async_copy(rtab_hbm.at[r_v.at[sl]], rows_v, sem).wait()
        pltpu.sync_copy(rows_v, out_hbm.at[rows, pl.ds(1, 1)])
        pltpu.async_copy(ptab_hbm.at[p_v.at[sl]], rows_v, sem).wait()
        pltpu.sync_copy(rows_v, out_hbm.at[rows, pl.ds(2, 1)])
        return carry

    lax.fori_loop(0, NCH, chunk, 0)


def kernel(node_types, trust_levels, providers, node_attrs, type_table,
           trust_table, provider_table, attr_W, attr_b):
    del node_attrs, attr_W, attr_b
    t = node_types.reshape(R).astype(jnp.int32)
    r = trust_levels.reshape(R).astype(jnp.int32)
    p = providers.reshape(R).astype(jnp.int32)
    ptab = jnp.pad(provider_table, ((0, 0), (0, DIM - PROVIDER_DIM)))
    out = _encode(t, r, p, type_table[:, None, :], trust_table[:, None, :],
                  ptab[:, None, :])
    return out.reshape(B, N, 3 * DIM)


# speedup vs baseline: 1.1994x; 1.0039x over previous
import functools

import jax
import jax.numpy as jnp
from jax import lax
from jax.experimental import pallas as pl
from jax.experimental.pallas import tpu as pltpu
from jax.experimental.pallas import tpu_sc as plsc

DIM = 128
PROVIDER_DIM = 64
B, N = 4096, 50
R = B * N
NC, NS = 2, 16
NW = NC * NS
RPW = R // NW
GC = 128
NCH = RPW // GC
NSLOT = 6
NBODY = (3 * NCH) // NSLOT

_mesh = plsc.VectorSubcoreMesh(
    core_axis_name="c", subcore_axis_name="s", num_cores=NC, num_subcores=NS)

_row_buf = pltpu.VMEM((GC, 1, DIM), jnp.float32)

@functools.partial(
    pl.kernel,
    out_type=jax.ShapeDtypeStruct((R, 3, DIM), jnp.float32),
    mesh=_mesh,
    scratch_types=(
        [pltpu.VMEM((RPW,), jnp.int32)] * 3
        + [_row_buf] * NSLOT
        + [pltpu.SemaphoreType.DMA] * (2 * NSLOT)
    ),
)
def _encode(types_hbm, trust_hbm, prov_hbm, ttab_hbm, rtab_hbm, ptab_hbm,
            out_hbm, t_v, r_v, p_v, *bufs_and_sems):
    rows = bufs_and_sems[:NSLOT]
    gsems = bufs_and_sems[NSLOT:2 * NSLOT]
    osems = bufs_and_sems[2 * NSLOT:]
    tabs = (ttab_hbm, rtab_hbm, ptab_hbm)
    idxs = (t_v, r_v, p_v)

    wid = lax.axis_index("s") * NC + lax.axis_index("c")
    base = wid * RPW
    pltpu.sync_copy(types_hbm.at[pl.ds(base, RPW)], t_v)
    pltpu.sync_copy(trust_hbm.at[pl.ds(base, RPW)], r_v)
    pltpu.sync_copy(prov_hbm.at[pl.ds(base, RPW)], p_v)

    def fire_gather(k, j):
        chunk = 2 * k + (j // 3)
        c = j % 3
        sl = pl.ds(chunk * GC, GC)
        return pltpu.async_copy(tabs[c].at[idxs[c].at[sl]], rows[j], gsems[j])

    def fire_write(k, j):
        chunk = 2 * k + (j // 3)
        c = j % 3
        dst = out_hbm.at[pl.ds(base + chunk * GC, GC), pl.ds(c, 1)]
        return pltpu.async_copy(rows[j], dst, osems[j])

    def wait_write(j):
        dst = out_hbm.at[pl.ds(0, GC), pl.ds(0, 1)]
        pltpu.make_async_copy(rows[j], dst, osems[j]).wait()

    def body(k, carry):
        @pl.when(k > 0)
        def _():
            for j in range(NSLOT):
                wait_write(j)

        handles = [fire_gather(k, j) for j in range(NSLOT)]
        for j in range(NSLOT):
            handles[j].wait()
            fire_write(k, j)
        return carry

    lax.fori_loop(0, NBODY, body, 0)
    for j in range(NSLOT):
        wait_write(j)


def kernel(node_types, trust_levels, providers, node_attrs, type_table,
           trust_table, provider_table, attr_W, attr_b):
    del node_attrs, attr_W, attr_b
    t = node_types.reshape(R).astype(jnp.int32)
    r = trust_levels.reshape(R).astype(jnp.int32)
    p = providers.reshape(R).astype(jnp.int32)
    ptab = jnp.pad(provider_table, ((0, 0), (0, DIM - PROVIDER_DIM)))
    out = _encode(t, r, p, type_table[:, None, :], trust_table[:, None, :],
                  ptab[:, None, :])
    return out.reshape(B, N, 3 * DIM)
